# baseline (speedup 1.0000x reference)
import functools
import math

import jax
import jax.numpy as jnp
from jax import lax
from jax.experimental import pallas as pl
from jax.experimental.pallas import tpu as pltpu
from jax.experimental.pallas import tpu_sc as plsc

VOCAB = 32768
SMOOTH = 0.1
CONF = 1.0 - SMOOTH
EPS = SMOOTH / (VOCAB - 2)
K_CONST = (VOCAB - 2) * EPS * math.log(EPS) + CONF * math.log(CONF)

N_ROWS = 4096
BLK_R = 64
BLK_V = 32768

SC_NC = 2
SC_NS = 16
SC_NW = SC_NC * SC_NS
SC_GCHUNK = N_ROWS // SC_NW


def _sc_body(xlin_hbm, tgt_hbm, out_hbm, tgt_v, idx_v, val_v, val0_v,
             stage_v, semg):
    wid = lax.axis_index("s") * SC_NC + lax.axis_index("c")
    gbase = wid * SC_GCHUNK
    pltpu.sync_copy(tgt_hbm.at[pl.ds(gbase, SC_GCHUNK)], tgt_v)
    for k in range(SC_GCHUNK // 16):
        t16 = tgt_v[pl.ds(k * 16, 16)]
        row = gbase + k * 16 + lax.iota(jnp.int32, 16)
        idx_v[pl.ds(k * 16, 16)] = (
            (row >> 3) * (VOCAB * 8) + (t16 >> 7) * 1024
            + (row & 7) * 128 + (t16 & 127))
    pltpu.async_copy(xlin_hbm.at[idx_v], val_v, semg).wait()
    for k in range(SC_GCHUNK // 16):
        row = gbase + k * 16 + lax.iota(jnp.int32, 16)
        idx_v[pl.ds(k * 16, 16)] = (row >> 3) * (VOCAB * 8) + (row & 7) * 128
    pltpu.async_copy(xlin_hbm.at[idx_v], val0_v, semg).wait()
    st = jnp.zeros((16,), jnp.float32)
    s0 = jnp.zeros((16,), jnp.float32)
    cw = jnp.zeros((16,), jnp.float32)
    for k in range(SC_GCHUNK // 16):
        t16 = tgt_v[pl.ds(k * 16, 16)]
        nz = t16 != 0
        st = st + jnp.where(nz, val_v[pl.ds(k * 16, 16)], 0.0)
        s0 = s0 + jnp.where(nz, val0_v[pl.ds(k * 16, 16)], 0.0)
        cw = cw + jnp.where(nz, 1.0, 0.0)
    stage_v[pl.ds(0, 16)] = st
    stage_v[pl.ds(16, 16)] = s0
    stage_v[pl.ds(32, 16)] = cw
    stage_v[pl.ds(48, 16)] = jnp.zeros((16,), jnp.float32)
    pltpu.sync_copy(stage_v, out_hbm.at[wid])


_sc_call = functools.partial(
    pl.kernel,
    out_type=jax.ShapeDtypeStruct((SC_NW, 64), jnp.float32),
    mesh=plsc.VectorSubcoreMesh(core_axis_name="c", subcore_axis_name="s"),
    scratch_types=[
        pltpu.VMEM((SC_GCHUNK,), jnp.int32),
        pltpu.VMEM((SC_GCHUNK,), jnp.int32),
        pltpu.VMEM((SC_GCHUNK,), jnp.float32),
        pltpu.VMEM((SC_GCHUNK,), jnp.float32),
        pltpu.VMEM((64,), jnp.float32),
        pltpu.SemaphoreType.DMA,
    ],
)(_sc_body)


def _tc_body(t_ref, x_ref, out_ref, acc_ref):
    i = pl.program_id(0)
    j = pl.program_id(1)
    ni = pl.num_programs(0)
    nj = pl.num_programs(1)

    @pl.when((i == 0) & (j == 0))
    def _init():
        acc_ref[0] = 0.0

    t = t_ref[...]
    w = (t != 0).astype(jnp.float32)
    xs = x_ref[...]
    rs = jnp.sum(xs, axis=1, keepdims=True)
    acc_ref[0] += jnp.sum(rs * w)

    @pl.when((i == ni - 1) & (j == nj - 1))
    def _fin():
        out_ref[0] = acc_ref[0]


def _combine_body(tc_ref, sc_ref, out_ref):
    blk = sc_ref[...]
    st = jnp.sum(blk[:, 0:16])
    s0 = jnp.sum(blk[:, 16:32])
    cnt = jnp.sum(blk[:, 32:48])
    out_ref[0] = (cnt * K_CONST - EPS * (tc_ref[0] - s0)
                  - (CONF - EPS) * st)


@jax.jit
def _loss(x, t32):
    x_lin = (x.reshape(N_ROWS // 8, 8, VOCAB // 128, 128)
             .transpose(0, 2, 1, 3).reshape(-1))
    sc_parts = _sc_call(x_lin, t32)

    grid = (N_ROWS // BLK_R, VOCAB // BLK_V)
    tc_part = pl.pallas_call(
        _tc_body,
        grid=grid,
        in_specs=[
            pl.BlockSpec((BLK_R, 1), lambda i, j: (i, 0)),
            pl.BlockSpec((BLK_R, BLK_V), lambda i, j: (i, j)),
        ],
        out_specs=pl.BlockSpec(memory_space=pltpu.SMEM),
        out_shape=jax.ShapeDtypeStruct((1,), jnp.float32),
        scratch_shapes=[pltpu.SMEM((1,), jnp.float32)],
    )(t32.reshape(-1, 1), x)

    res = pl.pallas_call(
        _combine_body,
        in_specs=[
            pl.BlockSpec(memory_space=pltpu.SMEM),
            pl.BlockSpec((SC_NW, 64), lambda: (0, 0)),
        ],
        out_specs=pl.BlockSpec(memory_space=pltpu.SMEM),
        out_shape=jax.ShapeDtypeStruct((1,), jnp.float32),
    )(tc_part, sc_parts)
    return res[0]


def kernel(x, target):
    return _loss(x, target.astype(jnp.int32))


# speedup vs baseline: 1.0032x; 1.0032x over previous
import functools
import math

import jax
import jax.numpy as jnp
from jax import lax
from jax.experimental import pallas as pl
from jax.experimental.pallas import tpu as pltpu
from jax.experimental.pallas import tpu_sc as plsc

VOCAB = 32768
SMOOTH = 0.1
CONF = 1.0 - SMOOTH
EPS = SMOOTH / (VOCAB - 2)
K_CONST = (VOCAB - 2) * EPS * math.log(EPS) + CONF * math.log(CONF)

N_ROWS = 4096
BLK_R = 128
BLK_V = 32768

SC_NC = 2
SC_NS = 16
SC_NW = SC_NC * SC_NS
SC_GCHUNK = N_ROWS // SC_NW


def _sc_body(xlin_hbm, tgt_hbm, out_hbm, tgt_v, idx_v, val_v, val0_v,
             stage_v, semg):
    wid = lax.axis_index("s") * SC_NC + lax.axis_index("c")
    gbase = wid * SC_GCHUNK
    pltpu.sync_copy(tgt_hbm.at[pl.ds(gbase, SC_GCHUNK)], tgt_v)
    for k in range(SC_GCHUNK // 16):
        t16 = tgt_v[pl.ds(k * 16, 16)]
        row = gbase + k * 16 + lax.iota(jnp.int32, 16)
        idx_v[pl.ds(k * 16, 16)] = (
            (row >> 3) * (VOCAB * 8) + (t16 >> 7) * 1024
            + (row & 7) * 128 + (t16 & 127))
    pltpu.async_copy(xlin_hbm.at[idx_v], val_v, semg).wait()
    for k in range(SC_GCHUNK // 16):
        row = gbase + k * 16 + lax.iota(jnp.int32, 16)
        idx_v[pl.ds(k * 16, 16)] = (row >> 3) * (VOCAB * 8) + (row & 7) * 128
    pltpu.async_copy(xlin_hbm.at[idx_v], val0_v, semg).wait()
    st = jnp.zeros((16,), jnp.float32)
    s0 = jnp.zeros((16,), jnp.float32)
    cw = jnp.zeros((16,), jnp.float32)
    for k in range(SC_GCHUNK // 16):
        t16 = tgt_v[pl.ds(k * 16, 16)]
        nz = t16 != 0
        st = st + jnp.where(nz, val_v[pl.ds(k * 16, 16)], 0.0)
        s0 = s0 + jnp.where(nz, val0_v[pl.ds(k * 16, 16)], 0.0)
        cw = cw + jnp.where(nz, 1.0, 0.0)
    stage_v[pl.ds(0, 16)] = st
    stage_v[pl.ds(16, 16)] = s0
    stage_v[pl.ds(32, 16)] = cw
    stage_v[pl.ds(48, 16)] = jnp.zeros((16,), jnp.float32)
    pltpu.sync_copy(stage_v, out_hbm.at[wid])


_sc_call = functools.partial(
    pl.kernel,
    out_type=jax.ShapeDtypeStruct((SC_NW, 64), jnp.float32),
    mesh=plsc.VectorSubcoreMesh(core_axis_name="c", subcore_axis_name="s"),
    scratch_types=[
        pltpu.VMEM((SC_GCHUNK,), jnp.int32),
        pltpu.VMEM((SC_GCHUNK,), jnp.int32),
        pltpu.VMEM((SC_GCHUNK,), jnp.float32),
        pltpu.VMEM((SC_GCHUNK,), jnp.float32),
        pltpu.VMEM((64,), jnp.float32),
        pltpu.SemaphoreType.DMA,
    ],
)(_sc_body)


def _tc_body(t_ref, x_ref, out_ref, acc_ref):
    i = pl.program_id(0)
    j = pl.program_id(1)
    ni = pl.num_programs(0)
    nj = pl.num_programs(1)

    @pl.when((i == 0) & (j == 0))
    def _init():
        acc_ref[0] = 0.0

    t = t_ref[...]
    w = (t != 0).astype(jnp.float32)
    xs = x_ref[...]
    rs = jnp.sum(xs, axis=1, keepdims=True)
    acc_ref[0] += jnp.sum(rs * w)

    @pl.when((i == ni - 1) & (j == nj - 1))
    def _fin():
        out_ref[0] = acc_ref[0]


def _combine_body(tc_ref, sc_ref, out_ref):
    blk = sc_ref[...]
    st = jnp.sum(blk[:, 0:16])
    s0 = jnp.sum(blk[:, 16:32])
    cnt = jnp.sum(blk[:, 32:48])
    out_ref[0] = (cnt * K_CONST - EPS * (tc_ref[0] - s0)
                  - (CONF - EPS) * st)


@jax.jit
def _loss(x, t32):
    x_lin = (x.reshape(N_ROWS // 8, 8, VOCAB // 128, 128)
             .transpose(0, 2, 1, 3).reshape(-1))
    sc_parts = _sc_call(x_lin, t32)

    grid = (N_ROWS // BLK_R, VOCAB // BLK_V)
    tc_part = pl.pallas_call(
        _tc_body,
        grid=grid,
        in_specs=[
            pl.BlockSpec((BLK_R, 1), lambda i, j: (i, 0)),
            pl.BlockSpec((BLK_R, BLK_V), lambda i, j: (i, j)),
        ],
        out_specs=pl.BlockSpec(memory_space=pltpu.SMEM),
        out_shape=jax.ShapeDtypeStruct((1,), jnp.float32),
        scratch_shapes=[pltpu.SMEM((1,), jnp.float32)],
    )(t32.reshape(-1, 1), x)

    res = pl.pallas_call(
        _combine_body,
        in_specs=[
            pl.BlockSpec(memory_space=pltpu.SMEM),
            pl.BlockSpec((SC_NW, 64), lambda: (0, 0)),
        ],
        out_specs=pl.BlockSpec(memory_space=pltpu.SMEM),
        out_shape=jax.ShapeDtypeStruct((1,), jnp.float32),
    )(tc_part, sc_parts)
    return res[0]


def kernel(x, target):
    return _loss(x, target.astype(jnp.int32))
